# baseline (device time: 71315 ns/iter reference)
import jax
import jax.numpy as jnp
from jax import lax
from jax.experimental import pallas as pl
from jax.experimental.pallas import tpu as pltpu

B, SQ, SKV, H, D = 8, 8, 1024, 16, 128
SCALE = D ** -0.5
ZSPLIT = 4
LEN = SKV // ZSPLIT


def _partial_body(q_hbm, k_hbm, v_hbm, acc_ref, l_ref,
                  qbuf, kbuf, vbuf, qsems, ksems, vsems):
    h = pl.program_id(0)
    kv0 = lax.axis_index("z") * LEN

    def copies(slot, hh):
        return (
            pltpu.make_async_copy(q_hbm.at[:, :, hh, :], qbuf.at[slot],
                                  qsems.at[slot]),
            pltpu.make_async_copy(k_hbm.at[:, pl.ds(kv0, LEN), hh, :],
                                  kbuf.at[slot], ksems.at[slot]),
            pltpu.make_async_copy(v_hbm.at[:, pl.ds(kv0, LEN), hh, :],
                                  vbuf.at[slot], vsems.at[slot]),
        )

    slot = lax.rem(h, 2)

    @pl.when(h == 0)
    def _():
        for c in copies(0, h):
            c.start()

    @pl.when(h + 1 < H)
    def _():
        for c in copies(lax.rem(h + 1, 2), h + 1):
            c.start()

    for c in copies(slot, h):
        c.wait()

    for b in range(B):
        q_b = qbuf[slot, b]
        k_b = kbuf[slot, b]
        v_b = vbuf[slot, b]
        s = lax.dot_general(
            q_b, k_b, dimension_numbers=(((1,), (1,)), ((), ())),
            preferred_element_type=jnp.float32,
        ) * SCALE
        p = jnp.exp(s)
        l_ref[h, b, :] = jnp.sum(p, axis=1)
        acc_ref[h, b] = lax.dot_general(
            p, v_b, dimension_numbers=(((1,), (0,)), ((), ())),
            preferred_element_type=jnp.float32,
        )


def _combine_body(acc_ref, l_ref, out_ref, cur_acc, cur_l,
                  recv_acc, recv_l, send_sems, recv_sems):
    my_x = lax.axis_index("x")
    my_y = lax.axis_index("y")
    my_z = lax.axis_index("z")

    cur_acc[...] = acc_ref[...]
    cur_l[...] = l_ref[...]

    partners = [
        (1 - my_x, my_y, my_z),
        (my_x, my_y, my_z ^ 1),
        (my_x, my_y, my_z ^ 2),
    ]
    for st, partner in enumerate(partners):
        rdma_acc = pltpu.make_async_remote_copy(
            src_ref=cur_acc, dst_ref=recv_acc.at[st],
            send_sem=send_sems.at[2 * st], recv_sem=recv_sems.at[2 * st],
            device_id=partner, device_id_type=pl.DeviceIdType.MESH,
        )
        rdma_l = pltpu.make_async_remote_copy(
            src_ref=cur_l, dst_ref=recv_l.at[st],
            send_sem=send_sems.at[2 * st + 1],
            recv_sem=recv_sems.at[2 * st + 1],
            device_id=partner, device_id_type=pl.DeviceIdType.MESH,
        )
        rdma_acc.start()
        rdma_l.start()
        rdma_acc.wait()
        rdma_l.wait()
        cur_acc[...] = cur_acc[...] + recv_acc[st]
        cur_l[...] = cur_l[...] + recv_l[st]

    for h in range(H):
        o_h = cur_acc[h] / cur_l[h][:, :, None]
        out_ref[:, :, h, :] = o_h


def kernel(Q, K, V):
    acc, l = pl.pallas_call(
        _partial_body,
        grid=(H,),
        in_specs=[
            pl.BlockSpec(memory_space=pl.ANY),
            pl.BlockSpec(memory_space=pl.ANY),
            pl.BlockSpec(memory_space=pl.ANY),
        ],
        out_specs=[
            pl.BlockSpec(memory_space=pltpu.VMEM),
            pl.BlockSpec(memory_space=pltpu.VMEM),
        ],
        out_shape=[
            jax.ShapeDtypeStruct((H, B, SQ, D), jnp.float32),
            jax.ShapeDtypeStruct((H, B, SQ), jnp.float32),
        ],
        scratch_shapes=[
            pltpu.VMEM((2, B, SQ, D), jnp.float32),
            pltpu.VMEM((2, B, LEN, D), jnp.float32),
            pltpu.VMEM((2, B, LEN, D), jnp.float32),
            pltpu.SemaphoreType.DMA((2,)),
            pltpu.SemaphoreType.DMA((2,)),
            pltpu.SemaphoreType.DMA((2,)),
        ],
    )(Q, K, V)

    out = pl.pallas_call(
        _combine_body,
        in_specs=[
            pl.BlockSpec(memory_space=pltpu.VMEM),
            pl.BlockSpec(memory_space=pltpu.VMEM),
        ],
        out_specs=pl.BlockSpec(memory_space=pltpu.VMEM),
        out_shape=jax.ShapeDtypeStruct((B, SQ, H, D), jnp.float32),
        scratch_shapes=[
            pltpu.VMEM((H, B, SQ, D), jnp.float32),
            pltpu.VMEM((H, B, SQ), jnp.float32),
            pltpu.VMEM((3, H, B, SQ, D), jnp.float32),
            pltpu.VMEM((3, H, B, SQ), jnp.float32),
            pltpu.SemaphoreType.DMA((6,)),
            pltpu.SemaphoreType.DMA((6,)),
        ],
    )(acc, l)
    return out
